# baseline (speedup 1.0000x reference)
import functools

import jax
import jax.numpy as jnp
from jax import lax
from jax.experimental import pallas as pl
from jax.experimental.pallas import tpu as pltpu
from jax.experimental.pallas import tpu_sc as plsc

_N = 10000
_E = 320000
_D = 128
_H = 128
_OUT = 2
_G = 64

_NC = 2
_NS = 16
_NW = _NC * _NS
_K = 128
_BLK = 8
_C = 80
_NB = _C // _BLK
_EPAD = _NW * _C * _K
_NP = 10240
_RPT = _NP // _NS

_mesh = plsc.VectorSubcoreMesh(core_axis_name="c", subcore_axis_name="s")


def _deg_body(dst_hbm, out_hbm, ones_v, idx_v, zrow_v, deg_sh):
    cid = lax.axis_index("c")
    sid = lax.axis_index("s")
    wid = sid * _NC + cid

    @pl.loop(0, _K // 16)
    def _fill_ones(i):
        ones_v[pl.ds(i * 16, 16)] = jnp.ones((16,), jnp.float32)

    @pl.loop(0, _RPT // 16)
    def _fill_zero(i):
        zrow_v[pl.ds(i * 16, 16)] = jnp.zeros((16,), jnp.float32)

    pltpu.sync_copy(zrow_v, deg_sh.at[pl.ds(sid * _RPT, _RPT)])
    plsc.subcore_barrier()

    @pl.loop(0, _C)
    def _chunk(c):
        base = (wid * _C + c) * _K
        pltpu.sync_copy(dst_hbm.at[pl.ds(base, _K)], idx_v)
        pltpu.sync_copy(ones_v, deg_sh.at[idx_v], add=True)

    plsc.subcore_barrier()
    pltpu.sync_copy(deg_sh.at[pl.ds(sid * _RPT, _RPT)],
                    out_hbm.at[cid, pl.ds(sid * _RPT, _RPT)])


_deg_call = pl.kernel(
    _deg_body,
    out_type=jax.ShapeDtypeStruct((_NC, _NP), jnp.float32),
    mesh=_mesh,
    scratch_types=[
        pltpu.VMEM((_K,), jnp.float32),
        pltpu.VMEM((_K,), jnp.int32),
        pltpu.VMEM((_RPT,), jnp.float32),
        pltpu.VMEM_SHARED((_NP,), jnp.float32),
    ],
)


def _agg_body(tbl_hbm, src_hbm, dst_hbm, out_hbm,
              idxs_v, idxd_v, rows0_v, rows1_v, acc_sh, sem0, sem1):
    cid = lax.axis_index("c")
    sid = lax.axis_index("s")
    wid = sid * _NC + cid

    @pl.loop(0, _K)
    def _zrow(i):
        @pl.loop(0, _D // 16)
        def _zcol(j):
            rows0_v[i, pl.ds(j * 16, 16)] = jnp.zeros((16,), jnp.float32)

    @pl.loop(0, _RPT // _K)
    def _zinit(r):
        pltpu.sync_copy(rows0_v, acc_sh.at[pl.ds(sid * _RPT + r * _K, _K)])

    plsc.subcore_barrier()

    bufs = (rows0_v, rows1_v)
    sems = (sem0, sem1)

    @pl.loop(0, _NB)
    def _block(b):
        row0 = wid * _C + b * _BLK
        pltpu.sync_copy(src_hbm.at[pl.ds(row0, _BLK)], idxs_v)
        pltpu.sync_copy(dst_hbm.at[pl.ds(row0, _BLK)], idxd_v)
        cps = [None] * _BLK
        cps[0] = pltpu.async_copy(tbl_hbm.at[idxs_v.at[0]], bufs[0], sems[0])
        for j in range(_BLK):
            if j + 1 < _BLK:
                cps[j + 1] = pltpu.async_copy(
                    tbl_hbm.at[idxs_v.at[j + 1]],
                    bufs[(j + 1) % 2], sems[(j + 1) % 2])
            cps[j].wait()
            pltpu.sync_copy(bufs[j % 2], acc_sh.at[idxd_v.at[j]], add=True)

    plsc.subcore_barrier()
    pltpu.sync_copy(acc_sh.at[pl.ds(sid * _RPT, _RPT)],
                    out_hbm.at[cid, pl.ds(sid * _RPT, _RPT)])


_agg_call = pl.kernel(
    _agg_body,
    out_type=jax.ShapeDtypeStruct((_NC, _NP, _D), jnp.float32),
    mesh=_mesh,
    scratch_types=[
        pltpu.VMEM((_BLK, _K), jnp.int32),
        pltpu.VMEM((_BLK, _K), jnp.int32),
        pltpu.VMEM((_K, _D), jnp.float32),
        pltpu.VMEM((_K, _D), jnp.float32),
        pltpu.VMEM_SHARED((_NP, _D), jnp.float32),
        pltpu.SemaphoreType.DMA,
        pltpu.SemaphoreType.DMA,
    ],
)


def _prep1_kernel(xp_ref, w1_ref, degp_ref, normed_ref, dinv_ref):
    deg = 1.0 + degp_ref[0] + degp_ref[1]
    dinv = 1.0 / jnp.sqrt(deg)
    xw = jnp.dot(xp_ref[...], w1_ref[...], preferred_element_type=jnp.float32)
    normed_ref[...] = xw * dinv
    dinv_ref[...] = dinv


def _prep1_call(xp, W1, degp):
    return pl.pallas_call(
        _prep1_kernel,
        out_shape=[
            jax.ShapeDtypeStruct((_NP, _D), jnp.float32),
            jax.ShapeDtypeStruct((_NP, 1), jnp.float32),
        ],
    )(xp, W1, degp)


def _post1_kernel(accp_ref, normed_ref, dinv_ref, b1_ref, g1_ref, be1_ref,
                  a1_ref, w2_ref, out_ref):
    dinv = dinv_ref[...]
    pre = dinv * (accp_ref[0] + accp_ref[1] + normed_ref[...]) + b1_ref[...]
    rows = lax.broadcasted_iota(jnp.int32, (_NP, 1), 0)
    maskf = (rows < _N).astype(jnp.float32)
    prem = pre * maskf
    m = jnp.sum(prem, axis=0, keepdims=True) / _N
    cen = (pre - m) * maskf
    v = jnp.sum(cen * cen, axis=0, keepdims=True) / _N
    bn = (pre - m) / jnp.sqrt(v + 1e-5) * g1_ref[...] + be1_ref[...]
    a1 = a1_ref[0, 0]
    h1 = jnp.where(bn >= 0, bn, a1 * bn) * maskf
    out_ref[...] = jnp.dot(h1, w2_ref[...],
                           preferred_element_type=jnp.float32) * dinv


def _post1_call(accp, normed, dinv, b1, g1, be1, a1, W2):
    return pl.pallas_call(
        _post1_kernel,
        out_shape=jax.ShapeDtypeStruct((_NP, _D), jnp.float32),
    )(accp, normed, dinv, b1, g1, be1, a1, W2)


def _final_kernel(accp_ref, normed_ref, dinv_ref, b2_ref, batch_ref,
                  wm1_ref, gm1_ref, bem1_ref, am_ref, wm2_ref, gm2_ref,
                  bem2_ref, wm3_ref, out_ref):
    h2 = dinv_ref[...] * (accp_ref[0] + accp_ref[1] + normed_ref[...]) \
        + b2_ref[...]
    gids = lax.broadcasted_iota(jnp.int32, (1, _G), 1)
    onehot = (batch_ref[...] == gids).astype(jnp.float32)
    s = lax.dot_general(onehot, h2, (((0,), (0,)), ((), ())),
                        preferred_element_type=jnp.float32,
                        precision=lax.Precision.HIGHEST)
    cnt = jnp.sum(onehot, axis=0)[:, None]
    h = s / jnp.maximum(cnt, 1.0)
    am = am_ref[0, 0]

    def mlp_bn(z, g, b):
        m = jnp.mean(z, axis=0, keepdims=True)
        v = jnp.mean((z - m) * (z - m), axis=0, keepdims=True)
        bn = (z - m) / jnp.sqrt(v + 1e-5) * g + b
        return jnp.where(bn >= 0, bn, am * bn)

    h = mlp_bn(jnp.dot(h, wm1_ref[...], preferred_element_type=jnp.float32),
               gm1_ref[...], bem1_ref[...])
    h = mlp_bn(jnp.dot(h, wm2_ref[...], preferred_element_type=jnp.float32),
               gm2_ref[...], bem2_ref[...])
    out_ref[...] = jnp.dot(h, wm3_ref[...],
                           preferred_element_type=jnp.float32)


def _final_call(accp, normed, dinv, b2, batchp, Wm1, gm1, bem1, am, Wm2,
                gm2, bem2, Wm3):
    return pl.pallas_call(
        _final_kernel,
        out_shape=jax.ShapeDtypeStruct((_G, _OUT), jnp.float32),
    )(accp, normed, dinv, b2, batchp, Wm1, gm1, bem1, am, Wm2, gm2, bem2,
      Wm3)


def kernel(x, edge_index, batch, W1, b1, g1, be1, a1, W2, b2,
           Wm1, gm1, bem1, am, Wm2, gm2, bem2, Wm3):
    src = edge_index[0]
    dst = edge_index[1]
    padi = jnp.full((_EPAD - _E,), _N, jnp.int32)
    srcp = jnp.concatenate([src, padi])
    dstp = jnp.concatenate([dst, padi])
    src2 = srcp.reshape(_EPAD // _K, _K)
    dst2 = dstp.reshape(_EPAD // _K, _K)
    xp = jnp.pad(x, ((0, _NP - _N), (0, 0)))
    batchp = jnp.pad(batch, (0, _NP - _N), constant_values=_G)
    batchp = batchp.reshape(_NP, 1)

    degp = _deg_call(dstp).reshape(_NC, _NP, 1)
    normed, dinv = _prep1_call(xp, W1, degp)
    accp = _agg_call(normed, src2, dst2)
    normed2 = _post1_call(accp, normed, dinv,
                          b1.reshape(1, _H), g1.reshape(1, _H),
                          be1.reshape(1, _H), a1.reshape(1, 1), W2)
    accp2 = _agg_call(normed2, src2, dst2)
    return _final_call(accp2, normed2, dinv, b2.reshape(1, _H), batchp,
                       Wm1, gm1.reshape(1, _H), bem1.reshape(1, _H),
                       am.reshape(1, 1), Wm2, gm2.reshape(1, _H),
                       bem2.reshape(1, _H), Wm3)


# speedup vs baseline: 1.6105x; 1.6105x over previous
import functools

import jax
import jax.numpy as jnp
from jax import lax
from jax.experimental import pallas as pl
from jax.experimental.pallas import tpu as pltpu
from jax.experimental.pallas import tpu_sc as plsc

_N = 10000
_E = 320000
_D = 128
_H = 128
_OUT = 2
_G = 64

_NC = 2
_NS = 16
_NW = _NC * _NS
_K = 128
_C = -(-_E // (_NW * _K))
_EPAD = _NW * _C * _K
_NP = 10240
_RPT = _NP // _NS

_mesh = plsc.VectorSubcoreMesh(core_axis_name="c", subcore_axis_name="s")


def _deg_body(dst_hbm, out_hbm, ones_v, idx_v, zrow_v, deg_sh):
    cid = lax.axis_index("c")
    sid = lax.axis_index("s")
    wid = sid * _NC + cid

    @pl.loop(0, _K // 16)
    def _fill_ones(i):
        ones_v[pl.ds(i * 16, 16)] = jnp.ones((16,), jnp.float32)

    @pl.loop(0, _RPT // 16)
    def _fill_zero(i):
        zrow_v[pl.ds(i * 16, 16)] = jnp.zeros((16,), jnp.float32)

    pltpu.sync_copy(zrow_v, deg_sh.at[pl.ds(sid * _RPT, _RPT)])
    plsc.subcore_barrier()

    @pl.loop(0, _C)
    def _chunk(c):
        base = (wid * _C + c) * _K
        pltpu.sync_copy(dst_hbm.at[pl.ds(base, _K)], idx_v)
        pltpu.sync_copy(ones_v, deg_sh.at[idx_v], add=True)

    plsc.subcore_barrier()
    pltpu.sync_copy(deg_sh.at[pl.ds(sid * _RPT, _RPT)],
                    out_hbm.at[cid, pl.ds(sid * _RPT, _RPT)])


_deg_call = pl.kernel(
    _deg_body,
    out_type=jax.ShapeDtypeStruct((_NC, _NP), jnp.float32),
    mesh=_mesh,
    scratch_types=[
        pltpu.VMEM((_K,), jnp.float32),
        pltpu.VMEM((_K,), jnp.int32),
        pltpu.VMEM((_RPT,), jnp.float32),
        pltpu.VMEM_SHARED((_NP,), jnp.float32),
    ],
)


def _agg_body(tbl_hbm, src_hbm, dst_hbm, out_hbm,
              idxs_v, idxd_v, rows_v, acc_sh, gsem):
    cid = lax.axis_index("c")
    sid = lax.axis_index("s")
    wid = sid * _NC + cid

    @pl.loop(0, _K)
    def _zrow(i):
        @pl.loop(0, _D // 16)
        def _zcol(j):
            rows_v[i, pl.ds(j * 16, 16)] = jnp.zeros((16,), jnp.float32)

    @pl.loop(0, _RPT // _K)
    def _zinit(r):
        pltpu.sync_copy(rows_v, acc_sh.at[pl.ds(sid * _RPT + r * _K, _K)])

    plsc.subcore_barrier()

    @pl.loop(0, _C)
    def _chunk(c):
        base = (wid * _C + c) * _K
        pltpu.sync_copy(src_hbm.at[pl.ds(base, _K)], idxs_v)
        pltpu.sync_copy(dst_hbm.at[pl.ds(base, _K)], idxd_v)
        pltpu.async_copy(tbl_hbm.at[idxs_v], rows_v, gsem).wait()
        pltpu.sync_copy(rows_v, acc_sh.at[idxd_v], add=True)

    plsc.subcore_barrier()
    pltpu.sync_copy(acc_sh.at[pl.ds(sid * _RPT, _RPT)],
                    out_hbm.at[cid, pl.ds(sid * _RPT, _RPT)])


_agg_call = pl.kernel(
    _agg_body,
    out_type=jax.ShapeDtypeStruct((_NC, _NP, _D), jnp.float32),
    mesh=_mesh,
    scratch_types=[
        pltpu.VMEM((_K,), jnp.int32),
        pltpu.VMEM((_K,), jnp.int32),
        pltpu.VMEM((_K, _D), jnp.float32),
        pltpu.VMEM_SHARED((_NP, _D), jnp.float32),
        pltpu.SemaphoreType.DMA,
    ],
)


def _prep1_kernel(xp_ref, w1_ref, degp_ref, normed_ref, dinv_ref):
    deg = 1.0 + degp_ref[0] + degp_ref[1]
    dinv = 1.0 / jnp.sqrt(deg)
    xw = jnp.dot(xp_ref[...], w1_ref[...], preferred_element_type=jnp.float32)
    normed_ref[...] = xw * dinv
    dinv_ref[...] = dinv


def _prep1_call(xp, W1, degp):
    return pl.pallas_call(
        _prep1_kernel,
        out_shape=[
            jax.ShapeDtypeStruct((_NP, _D), jnp.float32),
            jax.ShapeDtypeStruct((_NP, 1), jnp.float32),
        ],
    )(xp, W1, degp)


def _post1_kernel(accp_ref, normed_ref, dinv_ref, b1_ref, g1_ref, be1_ref,
                  a1_ref, w2_ref, out_ref):
    dinv = dinv_ref[...]
    pre = dinv * (accp_ref[0] + accp_ref[1] + normed_ref[...]) + b1_ref[...]
    rows = lax.broadcasted_iota(jnp.int32, (_NP, 1), 0)
    maskf = (rows < _N).astype(jnp.float32)
    prem = pre * maskf
    m = jnp.sum(prem, axis=0, keepdims=True) / _N
    cen = (pre - m) * maskf
    v = jnp.sum(cen * cen, axis=0, keepdims=True) / _N
    bn = (pre - m) / jnp.sqrt(v + 1e-5) * g1_ref[...] + be1_ref[...]
    a1 = a1_ref[0, 0]
    h1 = jnp.where(bn >= 0, bn, a1 * bn) * maskf
    out_ref[...] = jnp.dot(h1, w2_ref[...],
                           preferred_element_type=jnp.float32) * dinv


def _post1_call(accp, normed, dinv, b1, g1, be1, a1, W2):
    return pl.pallas_call(
        _post1_kernel,
        out_shape=jax.ShapeDtypeStruct((_NP, _D), jnp.float32),
    )(accp, normed, dinv, b1, g1, be1, a1, W2)


def _final_kernel(accp_ref, normed_ref, dinv_ref, b2_ref, batch_ref,
                  wm1_ref, gm1_ref, bem1_ref, am_ref, wm2_ref, gm2_ref,
                  bem2_ref, wm3_ref, out_ref):
    h2 = dinv_ref[...] * (accp_ref[0] + accp_ref[1] + normed_ref[...]) \
        + b2_ref[...]
    gids = lax.broadcasted_iota(jnp.int32, (1, _G), 1)
    onehot = (batch_ref[...] == gids).astype(jnp.float32)
    s = lax.dot_general(onehot, h2, (((0,), (0,)), ((), ())),
                        preferred_element_type=jnp.float32,
                        precision=lax.Precision.HIGHEST)
    cnt = jnp.sum(onehot, axis=0)[:, None]
    h = s / jnp.maximum(cnt, 1.0)
    am = am_ref[0, 0]

    def mlp_bn(z, g, b):
        m = jnp.mean(z, axis=0, keepdims=True)
        v = jnp.mean((z - m) * (z - m), axis=0, keepdims=True)
        bn = (z - m) / jnp.sqrt(v + 1e-5) * g + b
        return jnp.where(bn >= 0, bn, am * bn)

    h = mlp_bn(jnp.dot(h, wm1_ref[...], preferred_element_type=jnp.float32),
               gm1_ref[...], bem1_ref[...])
    h = mlp_bn(jnp.dot(h, wm2_ref[...], preferred_element_type=jnp.float32),
               gm2_ref[...], bem2_ref[...])
    out_ref[...] = jnp.dot(h, wm3_ref[...],
                           preferred_element_type=jnp.float32)


def _final_call(accp, normed, dinv, b2, batchp, Wm1, gm1, bem1, am, Wm2,
                gm2, bem2, Wm3):
    return pl.pallas_call(
        _final_kernel,
        out_shape=jax.ShapeDtypeStruct((_G, _OUT), jnp.float32),
    )(accp, normed, dinv, b2, batchp, Wm1, gm1, bem1, am, Wm2, gm2, bem2,
      Wm3)


def kernel(x, edge_index, batch, W1, b1, g1, be1, a1, W2, b2,
           Wm1, gm1, bem1, am, Wm2, gm2, bem2, Wm3):
    src = edge_index[0]
    dst = edge_index[1]
    padi = _N + (jnp.arange(_EPAD - _E, dtype=jnp.int32) % (_NP - _N))
    srcp = jnp.concatenate([src, padi])
    dstp = jnp.concatenate([dst, padi])
    xp = jnp.pad(x, ((0, _NP - _N), (0, 0)))
    batchp = jnp.pad(batch, (0, _NP - _N), constant_values=_G)
    batchp = batchp.reshape(_NP, 1)

    degp = _deg_call(dstp).reshape(_NC, _NP, 1)
    normed, dinv = _prep1_call(xp, W1, degp)
    accp = _agg_call(normed, srcp, dstp)
    normed2 = _post1_call(accp, normed, dinv,
                          b1.reshape(1, _H), g1.reshape(1, _H),
                          be1.reshape(1, _H), a1.reshape(1, 1), W2)
    accp2 = _agg_call(normed2, srcp, dstp)
    return _final_call(accp2, normed2, dinv, b2.reshape(1, _H), batchp,
                       Wm1, gm1.reshape(1, _H), bem1.reshape(1, _H),
                       am.reshape(1, 1), Wm2, gm2.reshape(1, _H),
                       bem2.reshape(1, _H), Wm3)


# speedup vs baseline: 2.1007x; 1.3044x over previous
import functools

import jax
import jax.numpy as jnp
from jax import lax
from jax.experimental import pallas as pl
from jax.experimental.pallas import tpu as pltpu
from jax.experimental.pallas import tpu_sc as plsc

_N = 10000
_E = 320000
_D = 128
_H = 128
_OUT = 2
_G = 64

_NC = 2
_NS = 16
_NW = _NC * _NS
_K = 128
_C = -(-_E // (_NW * _K))
_EPAD = _NW * _C * _K
_NP = 10240
_RPT = _NP // _NS

_mesh = plsc.VectorSubcoreMesh(core_axis_name="c", subcore_axis_name="s")


def _deg_body(dst_hbm, out_hbm, ones_v, idx_v, zrow_v, deg_sh):
    cid = lax.axis_index("c")
    sid = lax.axis_index("s")
    wid = sid * _NC + cid

    @pl.loop(0, _K // 16)
    def _fill_ones(i):
        ones_v[pl.ds(i * 16, 16)] = jnp.ones((16,), jnp.float32)

    @pl.loop(0, _RPT // 16)
    def _fill_zero(i):
        zrow_v[pl.ds(i * 16, 16)] = jnp.zeros((16,), jnp.float32)

    pltpu.sync_copy(zrow_v, deg_sh.at[pl.ds(sid * _RPT, _RPT)])
    plsc.subcore_barrier()

    @pl.loop(0, _C)
    def _chunk(c):
        base = (wid * _C + c) * _K
        pltpu.sync_copy(dst_hbm.at[pl.ds(base, _K)], idx_v)
        pltpu.sync_copy(ones_v, deg_sh.at[idx_v], add=True)

    plsc.subcore_barrier()
    pltpu.sync_copy(deg_sh.at[pl.ds(sid * _RPT, _RPT)],
                    out_hbm.at[cid, pl.ds(sid * _RPT, _RPT)])


_deg_call = pl.kernel(
    _deg_body,
    out_type=jax.ShapeDtypeStruct((_NC, _NP), jnp.float32),
    mesh=_mesh,
    scratch_types=[
        pltpu.VMEM((_K,), jnp.float32),
        pltpu.VMEM((_K,), jnp.int32),
        pltpu.VMEM((_RPT,), jnp.float32),
        pltpu.VMEM_SHARED((_NP,), jnp.float32),
    ],
)


def _agg_body(tbl_hbm, src_hbm, dst_hbm, out_hbm,
              idxsa_v, idxda_v, idxsb_v, idxdb_v, rowsa_v, rowsb_v,
              acc_sh, gsema, gsemb):
    cid = lax.axis_index("c")
    sid = lax.axis_index("s")
    wid = sid * _NC + cid

    @pl.loop(0, _K)
    def _zrow(i):
        @pl.loop(0, _D // 16)
        def _zcol(j):
            rowsa_v[i, pl.ds(j * 16, 16)] = jnp.zeros((16,), jnp.float32)

    @pl.loop(0, _RPT // _K)
    def _zinit(r):
        pltpu.sync_copy(rowsa_v, acc_sh.at[pl.ds(sid * _RPT + r * _K, _K)])

    plsc.subcore_barrier()

    @pl.loop(0, _C // 2)
    def _pair(p):
        basea = (wid * _C + 2 * p) * _K
        baseb = basea + _K
        pltpu.sync_copy(src_hbm.at[pl.ds(basea, _K)], idxsa_v)
        pltpu.sync_copy(dst_hbm.at[pl.ds(basea, _K)], idxda_v)
        cpa = pltpu.async_copy(tbl_hbm.at[idxsa_v], rowsa_v, gsema)
        pltpu.sync_copy(src_hbm.at[pl.ds(baseb, _K)], idxsb_v)
        pltpu.sync_copy(dst_hbm.at[pl.ds(baseb, _K)], idxdb_v)
        cpb = pltpu.async_copy(tbl_hbm.at[idxsb_v], rowsb_v, gsemb)
        cpa.wait()
        pltpu.sync_copy(rowsa_v, acc_sh.at[idxda_v], add=True)
        cpb.wait()
        pltpu.sync_copy(rowsb_v, acc_sh.at[idxdb_v], add=True)

    if _C % 2:
        base = (wid * _C + (_C - 1)) * _K
        pltpu.sync_copy(src_hbm.at[pl.ds(base, _K)], idxsa_v)
        pltpu.sync_copy(dst_hbm.at[pl.ds(base, _K)], idxda_v)
        pltpu.async_copy(tbl_hbm.at[idxsa_v], rowsa_v, gsema).wait()
        pltpu.sync_copy(rowsa_v, acc_sh.at[idxda_v], add=True)

    plsc.subcore_barrier()
    pltpu.sync_copy(acc_sh.at[pl.ds(sid * _RPT, _RPT)],
                    out_hbm.at[cid, pl.ds(sid * _RPT, _RPT)])


_agg_call = pl.kernel(
    _agg_body,
    out_type=jax.ShapeDtypeStruct((_NC, _NP, _D), jnp.float32),
    mesh=_mesh,
    scratch_types=[
        pltpu.VMEM((_K,), jnp.int32),
        pltpu.VMEM((_K,), jnp.int32),
        pltpu.VMEM((_K,), jnp.int32),
        pltpu.VMEM((_K,), jnp.int32),
        pltpu.VMEM((_K, _D), jnp.float32),
        pltpu.VMEM((_K, _D), jnp.float32),
        pltpu.VMEM_SHARED((_NP, _D), jnp.float32),
        pltpu.SemaphoreType.DMA,
        pltpu.SemaphoreType.DMA,
    ],
)


def _prep1_kernel(xp_ref, w1_ref, degp_ref, normed_ref, dinv_ref):
    deg = 1.0 + degp_ref[0] + degp_ref[1]
    dinv = 1.0 / jnp.sqrt(deg)
    xw = jnp.dot(xp_ref[...], w1_ref[...], preferred_element_type=jnp.float32)
    normed_ref[...] = xw * dinv
    dinv_ref[...] = dinv


def _prep1_call(xp, W1, degp):
    return pl.pallas_call(
        _prep1_kernel,
        out_shape=[
            jax.ShapeDtypeStruct((_NP, _D), jnp.float32),
            jax.ShapeDtypeStruct((_NP, 1), jnp.float32),
        ],
    )(xp, W1, degp)


def _post1_kernel(accp_ref, normed_ref, dinv_ref, b1_ref, g1_ref, be1_ref,
                  a1_ref, w2_ref, out_ref):
    dinv = dinv_ref[...]
    pre = dinv * (accp_ref[0] + accp_ref[1] + normed_ref[...]) + b1_ref[...]
    rows = lax.broadcasted_iota(jnp.int32, (_NP, 1), 0)
    maskf = (rows < _N).astype(jnp.float32)
    prem = pre * maskf
    m = jnp.sum(prem, axis=0, keepdims=True) / _N
    cen = (pre - m) * maskf
    v = jnp.sum(cen * cen, axis=0, keepdims=True) / _N
    bn = (pre - m) / jnp.sqrt(v + 1e-5) * g1_ref[...] + be1_ref[...]
    a1 = a1_ref[0, 0]
    h1 = jnp.where(bn >= 0, bn, a1 * bn) * maskf
    out_ref[...] = jnp.dot(h1, w2_ref[...],
                           preferred_element_type=jnp.float32) * dinv


def _post1_call(accp, normed, dinv, b1, g1, be1, a1, W2):
    return pl.pallas_call(
        _post1_kernel,
        out_shape=jax.ShapeDtypeStruct((_NP, _D), jnp.float32),
    )(accp, normed, dinv, b1, g1, be1, a1, W2)


def _final_kernel(accp_ref, normed_ref, dinv_ref, b2_ref, batch_ref,
                  wm1_ref, gm1_ref, bem1_ref, am_ref, wm2_ref, gm2_ref,
                  bem2_ref, wm3_ref, out_ref):
    h2 = dinv_ref[...] * (accp_ref[0] + accp_ref[1] + normed_ref[...]) \
        + b2_ref[...]
    gids = lax.broadcasted_iota(jnp.int32, (1, _G), 1)
    onehot = (batch_ref[...] == gids).astype(jnp.float32)
    s = lax.dot_general(onehot, h2, (((0,), (0,)), ((), ())),
                        preferred_element_type=jnp.float32,
                        precision=lax.Precision.HIGHEST)
    cnt = jnp.sum(onehot, axis=0)[:, None]
    h = s / jnp.maximum(cnt, 1.0)
    am = am_ref[0, 0]

    def mlp_bn(z, g, b):
        m = jnp.mean(z, axis=0, keepdims=True)
        v = jnp.mean((z - m) * (z - m), axis=0, keepdims=True)
        bn = (z - m) / jnp.sqrt(v + 1e-5) * g + b
        return jnp.where(bn >= 0, bn, am * bn)

    h = mlp_bn(jnp.dot(h, wm1_ref[...], preferred_element_type=jnp.float32),
               gm1_ref[...], bem1_ref[...])
    h = mlp_bn(jnp.dot(h, wm2_ref[...], preferred_element_type=jnp.float32),
               gm2_ref[...], bem2_ref[...])
    out_ref[...] = jnp.dot(h, wm3_ref[...],
                           preferred_element_type=jnp.float32)


def _final_call(accp, normed, dinv, b2, batchp, Wm1, gm1, bem1, am, Wm2,
                gm2, bem2, Wm3):
    return pl.pallas_call(
        _final_kernel,
        out_shape=jax.ShapeDtypeStruct((_G, _OUT), jnp.float32),
    )(accp, normed, dinv, b2, batchp, Wm1, gm1, bem1, am, Wm2, gm2, bem2,
      Wm3)


def kernel(x, edge_index, batch, W1, b1, g1, be1, a1, W2, b2,
           Wm1, gm1, bem1, am, Wm2, gm2, bem2, Wm3):
    src = edge_index[0]
    dst = edge_index[1]
    padi = _N + (jnp.arange(_EPAD - _E, dtype=jnp.int32) % (_NP - _N))
    srcp = jnp.concatenate([src, padi])
    dstp = jnp.concatenate([dst, padi])
    xp = jnp.pad(x, ((0, _NP - _N), (0, 0)))
    batchp = jnp.pad(batch, (0, _NP - _N), constant_values=_G)
    batchp = batchp.reshape(_NP, 1)

    degp = _deg_call(dstp).reshape(_NC, _NP, 1)
    normed, dinv = _prep1_call(xp, W1, degp)
    accp = _agg_call(normed, srcp, dstp)
    normed2 = _post1_call(accp, normed, dinv,
                          b1.reshape(1, _H), g1.reshape(1, _H),
                          be1.reshape(1, _H), a1.reshape(1, 1), W2)
    accp2 = _agg_call(normed2, srcp, dstp)
    return _final_call(accp2, normed2, dinv, b2.reshape(1, _H), batchp,
                       Wm1, gm1.reshape(1, _H), bem1.reshape(1, _H),
                       am.reshape(1, 1), Wm2, gm2.reshape(1, _H),
                       bem2.reshape(1, _H), Wm3)


# speedup vs baseline: 2.1814x; 1.0384x over previous
import functools

import jax
import jax.numpy as jnp
from jax import lax
from jax.experimental import pallas as pl
from jax.experimental.pallas import tpu as pltpu
from jax.experimental.pallas import tpu_sc as plsc

_N = 10000
_E = 320000
_D = 128
_H = 128
_OUT = 2
_G = 64

_NC = 2
_NS = 16
_NW = _NC * _NS
_K = 128
_C = -(-_E // (_NW * _K))
_EPAD = _NW * _C * _K
_NP = 10240
_RPT = _NP // _NS

_mesh = plsc.VectorSubcoreMesh(core_axis_name="c", subcore_axis_name="s")


def _deg_body(dst_hbm, out_hbm, ones_v, idxa_v, idxb_v, zrow_v, deg_sh,
              dsema, dsemb):
    cid = lax.axis_index("c")
    sid = lax.axis_index("s")
    wid = sid * _NC + cid

    @pl.loop(0, _K // 16)
    def _fill_ones(i):
        ones_v[pl.ds(i * 16, 16)] = jnp.ones((16,), jnp.float32)

    @pl.loop(0, _RPT // 16)
    def _fill_zero(i):
        zrow_v[pl.ds(i * 16, 16)] = jnp.zeros((16,), jnp.float32)

    pltpu.sync_copy(zrow_v, deg_sh.at[pl.ds(sid * _RPT, _RPT)])
    plsc.subcore_barrier()

    @pl.loop(0, _C // 2)
    def _pair(p):
        basea = (wid * _C + 2 * p) * _K
        cpa = pltpu.async_copy(dst_hbm.at[pl.ds(basea, _K)], idxa_v, dsema)
        cpb = pltpu.async_copy(dst_hbm.at[pl.ds(basea + _K, _K)], idxb_v,
                               dsemb)
        cpa.wait()
        pltpu.sync_copy(ones_v, deg_sh.at[idxa_v], add=True)
        cpb.wait()
        pltpu.sync_copy(ones_v, deg_sh.at[idxb_v], add=True)

    if _C % 2:
        base = (wid * _C + (_C - 1)) * _K
        pltpu.sync_copy(dst_hbm.at[pl.ds(base, _K)], idxa_v)
        pltpu.sync_copy(ones_v, deg_sh.at[idxa_v], add=True)

    plsc.subcore_barrier()
    pltpu.sync_copy(deg_sh.at[pl.ds(sid * _RPT, _RPT)],
                    out_hbm.at[cid, pl.ds(sid * _RPT, _RPT)])


_deg_call = pl.kernel(
    _deg_body,
    out_type=jax.ShapeDtypeStruct((_NC, _NP), jnp.float32),
    mesh=_mesh,
    scratch_types=[
        pltpu.VMEM((_K,), jnp.float32),
        pltpu.VMEM((_K,), jnp.int32),
        pltpu.VMEM((_K,), jnp.int32),
        pltpu.VMEM((_RPT,), jnp.float32),
        pltpu.VMEM_SHARED((_NP,), jnp.float32),
        pltpu.SemaphoreType.DMA,
        pltpu.SemaphoreType.DMA,
    ],
)


def _agg_body(tbl_hbm, src_hbm, dst_hbm, out_hbm,
              idxsa_v, idxda_v, idxsb_v, idxdb_v, rowsa_v, rowsb_v,
              acc_sh, gsema, gsemb):
    cid = lax.axis_index("c")
    sid = lax.axis_index("s")
    wid = sid * _NC + cid

    @pl.loop(0, _K)
    def _zrow(i):
        @pl.loop(0, _D // 16)
        def _zcol(j):
            rowsa_v[i, pl.ds(j * 16, 16)] = jnp.zeros((16,), jnp.float32)

    @pl.loop(0, _RPT // _K)
    def _zinit(r):
        pltpu.sync_copy(rowsa_v, acc_sh.at[pl.ds(sid * _RPT + r * _K, _K)])

    plsc.subcore_barrier()

    @pl.loop(0, _C // 2)
    def _pair(p):
        basea = (wid * _C + 2 * p) * _K
        baseb = basea + _K
        pltpu.sync_copy(src_hbm.at[pl.ds(basea, _K)], idxsa_v)
        pltpu.sync_copy(dst_hbm.at[pl.ds(basea, _K)], idxda_v)
        cpa = pltpu.async_copy(tbl_hbm.at[idxsa_v], rowsa_v, gsema)
        pltpu.sync_copy(src_hbm.at[pl.ds(baseb, _K)], idxsb_v)
        pltpu.sync_copy(dst_hbm.at[pl.ds(baseb, _K)], idxdb_v)
        cpb = pltpu.async_copy(tbl_hbm.at[idxsb_v], rowsb_v, gsemb)
        cpa.wait()
        pltpu.sync_copy(rowsa_v, acc_sh.at[idxda_v], add=True)
        cpb.wait()
        pltpu.sync_copy(rowsb_v, acc_sh.at[idxdb_v], add=True)

    if _C % 2:
        base = (wid * _C + (_C - 1)) * _K
        pltpu.sync_copy(src_hbm.at[pl.ds(base, _K)], idxsa_v)
        pltpu.sync_copy(dst_hbm.at[pl.ds(base, _K)], idxda_v)
        pltpu.async_copy(tbl_hbm.at[idxsa_v], rowsa_v, gsema).wait()
        pltpu.sync_copy(rowsa_v, acc_sh.at[idxda_v], add=True)

    plsc.subcore_barrier()
    pltpu.sync_copy(acc_sh.at[pl.ds(sid * _RPT, _RPT)],
                    out_hbm.at[cid, pl.ds(sid * _RPT, _RPT)])


_agg_call = pl.kernel(
    _agg_body,
    out_type=jax.ShapeDtypeStruct((_NC, _NP, _D), jnp.float32),
    mesh=_mesh,
    scratch_types=[
        pltpu.VMEM((_K,), jnp.int32),
        pltpu.VMEM((_K,), jnp.int32),
        pltpu.VMEM((_K,), jnp.int32),
        pltpu.VMEM((_K,), jnp.int32),
        pltpu.VMEM((_K, _D), jnp.float32),
        pltpu.VMEM((_K, _D), jnp.float32),
        pltpu.VMEM_SHARED((_NP, _D), jnp.float32),
        pltpu.SemaphoreType.DMA,
        pltpu.SemaphoreType.DMA,
    ],
)


def _xw_kernel(xp_ref, w1_ref, xw_ref):
    xw_ref[...] = jnp.dot(xp_ref[...], w1_ref[...],
                          preferred_element_type=jnp.float32)


def _xw_call(xp, W1):
    return pl.pallas_call(
        _xw_kernel,
        out_shape=jax.ShapeDtypeStruct((_NP, _D), jnp.float32),
    )(xp, W1)


def _prep1_kernel(xw_ref, degp_ref, normed_ref, dinv_ref):
    deg = 1.0 + degp_ref[0] + degp_ref[1]
    dinv = 1.0 / jnp.sqrt(deg)
    normed_ref[...] = xw_ref[...] * dinv
    dinv_ref[...] = dinv


def _prep1_call(xw, degp):
    return pl.pallas_call(
        _prep1_kernel,
        out_shape=[
            jax.ShapeDtypeStruct((_NP, _D), jnp.float32),
            jax.ShapeDtypeStruct((_NP, 1), jnp.float32),
        ],
    )(xw, degp)


def _post1_kernel(accp_ref, normed_ref, dinv_ref, b1_ref, g1_ref, be1_ref,
                  a1_ref, w2_ref, out_ref):
    dinv = dinv_ref[...]
    pre = dinv * (accp_ref[0] + accp_ref[1] + normed_ref[...]) + b1_ref[...]
    rows = lax.broadcasted_iota(jnp.int32, (_NP, 1), 0)
    maskf = (rows < _N).astype(jnp.float32)
    prem = pre * maskf
    m = jnp.sum(prem, axis=0, keepdims=True) / _N
    cen = (pre - m) * maskf
    v = jnp.sum(cen * cen, axis=0, keepdims=True) / _N
    bn = (pre - m) / jnp.sqrt(v + 1e-5) * g1_ref[...] + be1_ref[...]
    a1 = a1_ref[0, 0]
    h1 = jnp.where(bn >= 0, bn, a1 * bn) * maskf
    out_ref[...] = jnp.dot(h1, w2_ref[...],
                           preferred_element_type=jnp.float32) * dinv


def _post1_call(accp, normed, dinv, b1, g1, be1, a1, W2):
    return pl.pallas_call(
        _post1_kernel,
        out_shape=jax.ShapeDtypeStruct((_NP, _D), jnp.float32),
    )(accp, normed, dinv, b1, g1, be1, a1, W2)


def _final_kernel(accp_ref, normed_ref, dinv_ref, b2_ref, batch_ref,
                  wm1_ref, gm1_ref, bem1_ref, am_ref, wm2_ref, gm2_ref,
                  bem2_ref, wm3_ref, out_ref):
    h2 = dinv_ref[...] * (accp_ref[0] + accp_ref[1] + normed_ref[...]) \
        + b2_ref[...]
    gids = lax.broadcasted_iota(jnp.int32, (1, _G), 1)
    onehot = (batch_ref[...] == gids).astype(jnp.float32)
    s = lax.dot_general(onehot, h2, (((0,), (0,)), ((), ())),
                        preferred_element_type=jnp.float32,
                        precision=lax.Precision.HIGHEST)
    cnt = jnp.sum(onehot, axis=0)[:, None]
    h = s / jnp.maximum(cnt, 1.0)
    am = am_ref[0, 0]

    def mlp_bn(z, g, b):
        m = jnp.mean(z, axis=0, keepdims=True)
        v = jnp.mean((z - m) * (z - m), axis=0, keepdims=True)
        bn = (z - m) / jnp.sqrt(v + 1e-5) * g + b
        return jnp.where(bn >= 0, bn, am * bn)

    h = mlp_bn(jnp.dot(h, wm1_ref[...], preferred_element_type=jnp.float32),
               gm1_ref[...], bem1_ref[...])
    h = mlp_bn(jnp.dot(h, wm2_ref[...], preferred_element_type=jnp.float32),
               gm2_ref[...], bem2_ref[...])
    out_ref[...] = jnp.dot(h, wm3_ref[...],
                           preferred_element_type=jnp.float32)


def _final_call(accp, normed, dinv, b2, batchp, Wm1, gm1, bem1, am, Wm2,
                gm2, bem2, Wm3):
    return pl.pallas_call(
        _final_kernel,
        out_shape=jax.ShapeDtypeStruct((_G, _OUT), jnp.float32),
    )(accp, normed, dinv, b2, batchp, Wm1, gm1, bem1, am, Wm2, gm2, bem2,
      Wm3)


def kernel(x, edge_index, batch, W1, b1, g1, be1, a1, W2, b2,
           Wm1, gm1, bem1, am, Wm2, gm2, bem2, Wm3):
    src = edge_index[0]
    dst = edge_index[1]
    padi = _N + (jnp.arange(_EPAD - _E, dtype=jnp.int32) % (_NP - _N))
    srcp = jnp.concatenate([src, padi])
    dstp = jnp.concatenate([dst, padi])
    xp = jnp.pad(x, ((0, _NP - _N), (0, 0)))
    batchp = jnp.pad(batch, (0, _NP - _N), constant_values=_G)
    batchp = batchp.reshape(_NP, 1)

    xw = _xw_call(xp, W1)
    degp = _deg_call(dstp).reshape(_NC, _NP, 1)
    normed, dinv = _prep1_call(xw, degp)
    accp = _agg_call(normed, srcp, dstp)
    normed2 = _post1_call(accp, normed, dinv,
                          b1.reshape(1, _H), g1.reshape(1, _H),
                          be1.reshape(1, _H), a1.reshape(1, 1), W2)
    accp2 = _agg_call(normed2, srcp, dstp)
    return _final_call(accp2, normed2, dinv, b2.reshape(1, _H), batchp,
                       Wm1, gm1.reshape(1, _H), bem1.reshape(1, _H),
                       am.reshape(1, 1), Wm2, gm2.reshape(1, _H),
                       bem2.reshape(1, _H), Wm3)
